# baseline (speedup 1.0000x reference)
import functools

import jax
import jax.numpy as jnp
from jax import lax
from jax.experimental import pallas as pl
from jax.experimental.pallas import tpu as pltpu
from jax.experimental.pallas import tpu_sc as plsc

N = 10000
E = 320000
D = 128
NC = 2
NS = 16
NW = NC * NS
NP = 10240
RPT_DEG = NP // NS
EPT2 = E // NS
K = 80
EPT = E // NW
CH = EPT // K
R = 5040
NPASS = 2
ACC = 5120
DUMP = R

f32 = jnp.float32

_mesh = plsc.VectorSubcoreMesh(
    core_axis_name="c", subcore_axis_name="s", num_cores=NC, num_subcores=NS)
_sc_params = pltpu.CompilerParams(needs_layout_passes=False)


LSZ2 = 10240


@functools.partial(
    pl.kernel,
    out_type=[
        jax.ShapeDtypeStruct((NW, NP), f32),
        jax.ShapeDtypeStruct((2, NP), f32),
        jax.ShapeDtypeStruct((NW, 4, LSZ2), jnp.int32),
        jax.ShapeDtypeStruct((NW, 16), jnp.int32),
    ],
    mesh=_mesh,
    scratch_types=[
        pltpu.VMEM((EPT2,), jnp.int32),
        pltpu.VMEM((NP,), f32),
        pltpu.VMEM((RPT_DEG,), f32),
        pltpu.VMEM((RPT_DEG,), f32),
        pltpu.VMEM((EPT,), jnp.int32),
        pltpu.VMEM((EPT,), jnp.int32),
        pltpu.VMEM((LSZ2,), jnp.int32),
        pltpu.VMEM((LSZ2,), jnp.int32),
        pltpu.VMEM((LSZ2,), jnp.int32),
        pltpu.VMEM((LSZ2,), jnp.int32),
        pltpu.VMEM((16,), jnp.int32),
    ],
    compiler_params=_sc_params,
)
def _prep_kernel(idx_hbm, csrc_hbm, cdst_hbm,
                 part_out, deg_out, lists_out, cnt_out,
                 idx_v, deg_v, acc_v, tmp_v, csrc_v, cdst_v,
                 ls0, ld0, ls1, ld1, cnt_v):
    c = lax.axis_index("c")
    s = lax.axis_index("s")
    row = c * NS + s
    pltpu.sync_copy(idx_hbm.at[row], idx_v)
    pltpu.sync_copy(csrc_hbm.at[row], csrc_v)
    pltpu.sync_copy(cdst_hbm.at[row], cdst_v)

    r1 = jnp.full((16,), R, jnp.int32)

    def cpbody(i, cnts):
        c0, c1 = cnts
        sl = pl.ds(i * 16, 16)
        s16 = csrc_v[sl]
        d16 = cdst_v[sl]
        m0 = d16 < r1
        m1 = d16 >= r1
        plsc.store_compressed(ls0.at[pl.ds(c0, 16)], s16, mask=m0)
        plsc.store_compressed(ld0.at[pl.ds(c0, 16)], d16, mask=m0)
        plsc.store_compressed(ls1.at[pl.ds(c1, 16)], s16, mask=m1)
        plsc.store_compressed(ld1.at[pl.ds(c1, 16)], d16 - r1, mask=m1)
        n0 = plsc.all_reduce_population_count(m0)[0]
        return (c0 + n0, c1 + (16 - n0))

    zi = jnp.int32(0)
    cnt0, cnt1 = lax.fori_loop(0, EPT // 16, cpbody, (zi, zi))

    zero16i = jnp.zeros((16,), jnp.int32)
    dump16 = jnp.full((16,), DUMP, jnp.int32) + (
        lax.iota(jnp.int32, 16) & jnp.full((16,), 63, jnp.int32))
    for cn, slist, dlist in ((cnt0, ls0, ld0), (cnt1, ls1, ld1)):
        for t in range(15):
            slist[pl.ds(cn + t * 16, 16)] = zero16i
            dlist[pl.ds(cn + t * 16, 16)] = dump16

    iota16 = lax.iota(jnp.int32, 16)
    cv = jnp.where(iota16 == 0, jnp.full((16,), cnt0, jnp.int32),
                   jnp.where(iota16 == 1, jnp.full((16,), cnt1, jnp.int32),
                             jnp.zeros((16,), jnp.int32)))
    cnt_v[...] = cv
    pltpu.sync_copy(cnt_v, cnt_out.at[row])
    pltpu.sync_copy(ls0, lists_out.at[row, 0])
    pltpu.sync_copy(ld0, lists_out.at[row, 1])
    pltpu.sync_copy(ls1, lists_out.at[row, 2])
    pltpu.sync_copy(ld1, lists_out.at[row, 3])

    zero16 = jnp.zeros((16,), f32)
    ones16 = jnp.ones((16,), f32)

    def zbody(i, carry):
        deg_v[pl.ds(i * 16, 16)] = zero16
        return carry
    lax.fori_loop(0, NP // 16, zbody, None)

    def ebody(e, carry):
        idx = idx_v[pl.ds(e * 16, 16)]
        plsc.addupdate_scatter(deg_v, [idx], ones16)
        return carry
    lax.fori_loop(0, EPT2 // 16, ebody, None)

    pltpu.sync_copy(deg_v, part_out.at[row])
    plsc.subcore_barrier()

    base = s * RPT_DEG
    pltpu.sync_copy(part_out.at[c * NS, pl.ds(base, RPT_DEG)], acc_v)
    for p in range(1, NS):
        pltpu.sync_copy(part_out.at[c * NS + p, pl.ds(base, RPT_DEG)], tmp_v)

        def abody(i, carry):
            sl = pl.ds(i * 16, 16)
            acc_v[sl] = acc_v[sl] + tmp_v[sl]
            return carry
        lax.fori_loop(0, RPT_DEG // 16, abody, None)
    pltpu.sync_copy(acc_v, deg_out.at[c, pl.ds(base, RPT_DEG)])


@functools.partial(
    pl.kernel,
    out_type=jax.ShapeDtypeStruct((NC, N, D), f32),
    mesh=_mesh,
    scratch_types=[
        pltpu.VMEM((LSZ2,), jnp.int32),
        pltpu.VMEM((LSZ2,), jnp.int32),
        pltpu.VMEM((LSZ2,), jnp.int32),
        pltpu.VMEM((LSZ2,), jnp.int32),
        pltpu.VMEM((16,), jnp.int32),
        pltpu.VMEM((K, D), f32),
        pltpu.VMEM((K, D), f32),
        pltpu.VMEM((K, D), f32),
        pltpu.VMEM_SHARED((ACC, D), f32),
        pltpu.SemaphoreType.DMA,
        pltpu.SemaphoreType.DMA,
    ],
    compiler_params=_sc_params,
)
def _scatter_kernel(lists_hbm, cnt_hbm, h_hbm, out_hbm,
                    ls0, ld0, ls1, ld1, cnt_v, rows_a, rows_b, zbuf,
                    acc_sh, sem_a, sem_b):
    c = lax.axis_index("c")
    s = lax.axis_index("s")
    w = c * NS + s
    pltpu.sync_copy(lists_hbm.at[w, 0], ls0)
    pltpu.sync_copy(lists_hbm.at[w, 1], ld0)
    pltpu.sync_copy(lists_hbm.at[w, 2], ls1)
    pltpu.sync_copy(lists_hbm.at[w, 3], ld1)
    pltpu.sync_copy(cnt_hbm.at[w], cnt_v)
    cnts = cnt_v[...]
    cnt0 = cnts[0]
    cnt1 = cnts[1]

    zero16 = jnp.zeros((16,), f32)

    def zrow(i, carry):
        for j in range(D // 16):
            zbuf[i, pl.ds(j * 16, 16)] = zero16
        return carry
    lax.fori_loop(0, K, zrow, None)

    def zero_acc():
        for i in range(-(-(ACC // K) // NS)):
            m = i * NS + s

            @pl.when(m < ACC // K)
            def _():
                pltpu.sync_copy(zbuf, acc_sh.at[pl.ds(m * K, K)])

    zero_acc()
    plsc.subcore_barrier()

    for p, (cn, slist, dlist) in enumerate(((cnt0, ls0, ld0),
                                            (cnt1, ls1, ld1))):
        pairs = jnp.maximum((cn + 2 * K - 1) // (2 * K), 1)
        pltpu.async_copy(h_hbm.at[slist.at[pl.ds(0, K)]], rows_a, sem_a)

        def pair(j2, carry, slist=slist, dlist=dlist):
            j = j2 * 2 * K
            pltpu.make_async_copy(
                h_hbm.at[slist.at[pl.ds(j, K)]], rows_a, sem_a).wait()
            pltpu.async_copy(
                h_hbm.at[slist.at[pl.ds(j + K, K)]], rows_b, sem_b)
            pltpu.sync_copy(rows_a, acc_sh.at[dlist.at[pl.ds(j, K)]],
                            add=True)
            pltpu.make_async_copy(
                h_hbm.at[slist.at[pl.ds(j + K, K)]], rows_b, sem_b).wait()
            pltpu.async_copy(
                h_hbm.at[slist.at[pl.ds(j + 2 * K, K)]], rows_a, sem_a)
            pltpu.sync_copy(rows_b, acc_sh.at[dlist.at[pl.ds(j + K, K)]],
                            add=True)
            return carry
        lax.fori_loop(0, pairs, pair, None)
        pltpu.make_async_copy(
            h_hbm.at[slist.at[pl.ds(0, K)]], rows_a, sem_a).wait()

        plsc.subcore_barrier()

        rp = min(R, N - p * R)
        cp = rp // K
        for i in range(-(-cp // NS)):
            m = i * NS + s

            @pl.when(m < cp)
            def _(m=m):
                pltpu.sync_copy(acc_sh.at[pl.ds(m * K, K)], zbuf)
                pltpu.sync_copy(zbuf, out_hbm.at[c, pl.ds(p * R + m * K, K)])

        if p < NPASS - 1:
            lax.fori_loop(0, K, zrow, None)
            zero_acc()
            plsc.subcore_barrier()


_BLK = 2000
_GRID = N // _BLK


def _mm_scale_body(x_ref, w_ref, b_ref, degs_ref, o_ref):
    h = jnp.dot(x_ref[...], w_ref[...], preferred_element_type=f32) + b_ref[...]
    o_ref[...] = h * lax.rsqrt(jnp.maximum(degs_ref[...], 1.0))


def _tc_mm_scale(x, w, b2d, degs):
    return pl.pallas_call(
        _mm_scale_body,
        grid=(_GRID,),
        in_specs=[
            pl.BlockSpec((_BLK, D), lambda i: (i, 0)),
            pl.BlockSpec((D, D), lambda i: (0, 0)),
            pl.BlockSpec((1, D), lambda i: (0, 0)),
            pl.BlockSpec((_BLK, 1), lambda i: (i, 0)),
        ],
        out_specs=pl.BlockSpec((_BLK, D), lambda i: (i, 0)),
        out_shape=jax.ShapeDtypeStruct((N, D), f32),
    )(x, w, b2d, degs)


def _mid_body(p_ref, degd_ref, w_ref, b_ref, degs_ref, x1_ref, h2_ref):
    a = (p_ref[0] + p_ref[1]) * lax.rsqrt(jnp.maximum(degd_ref[...], 1.0))
    x1 = jnp.where(a >= 0, a, 0.01 * a)
    x1_ref[...] = x1
    h2 = jnp.dot(x1, w_ref[...], preferred_element_type=f32) + b_ref[...]
    h2_ref[...] = h2 * lax.rsqrt(jnp.maximum(degs_ref[...], 1.0))


def _tc_mid(p, degd, w, b2d, degs):
    return pl.pallas_call(
        _mid_body,
        grid=(_GRID,),
        in_specs=[
            pl.BlockSpec((NC, _BLK, D), lambda i: (0, i, 0)),
            pl.BlockSpec((_BLK, 1), lambda i: (i, 0)),
            pl.BlockSpec((D, D), lambda i: (0, 0)),
            pl.BlockSpec((1, D), lambda i: (0, 0)),
            pl.BlockSpec((_BLK, 1), lambda i: (i, 0)),
        ],
        out_specs=[
            pl.BlockSpec((_BLK, D), lambda i: (i, 0)),
            pl.BlockSpec((_BLK, D), lambda i: (i, 0)),
        ],
        out_shape=[
            jax.ShapeDtypeStruct((N, D), f32),
            jax.ShapeDtypeStruct((N, D), f32),
        ],
    )(p, degd, w, b2d, degs)


def _fin_body(q_ref, degd_ref, x0_ref, x1_ref, o_ref):
    a = (q_ref[0] + q_ref[1]) * lax.rsqrt(jnp.maximum(degd_ref[...], 1.0))
    x2 = jnp.where(a >= 0, a, 0.01 * a)
    o_ref[...] = (x0_ref[...] + x1_ref[...] + x2) * (1.0 / 3.0)


def _tc_fin(q, degd, x0, x1):
    return pl.pallas_call(
        _fin_body,
        grid=(_GRID,),
        in_specs=[
            pl.BlockSpec((NC, _BLK, D), lambda i: (0, i, 0)),
            pl.BlockSpec((_BLK, 1), lambda i: (i, 0)),
            pl.BlockSpec((_BLK, D), lambda i: (i, 0)),
            pl.BlockSpec((_BLK, D), lambda i: (i, 0)),
        ],
        out_specs=pl.BlockSpec((_BLK, D), lambda i: (i, 0)),
        out_shape=jax.ShapeDtypeStruct((N, D), f32),
    )(q, degd, x0, x1)


def kernel(edge_index, all_embed, W1, b1, W2, b2):
    ei = edge_index.astype(jnp.int32)
    deg_idx = ei.reshape(NW, EPT2)
    src_r = ei[0].reshape(NW, EPT)
    dst_r = ei[1].reshape(NW, EPT)

    _, degs, lists, cnts = _prep_kernel(deg_idx, src_r, dst_r)
    deg_src = degs[0, :N].reshape(N, 1)
    deg_dst = degs[1, :N].reshape(N, 1)
    b1r = b1.reshape(1, D)
    b2r = b2.reshape(1, D)

    h1 = _tc_mm_scale(all_embed, W1, b1r, deg_src)
    p = _scatter_kernel(lists, cnts, h1)
    x1, h2 = _tc_mid(p, deg_dst, W2, b2r, deg_src)
    q = _scatter_kernel(lists, cnts, h2)
    return _tc_fin(q, deg_dst, all_embed, x1)


# speedup vs baseline: 2.9873x; 2.9873x over previous
import functools

import jax
import jax.numpy as jnp
from jax import lax
from jax.experimental import pallas as pl
from jax.experimental.pallas import tpu as pltpu
from jax.experimental.pallas import tpu_sc as plsc

N = 10000
E = 320000
D = 128
NC = 2
NS = 16
NW = NC * NS
NP = 10240
RPT_DEG = NP // NS
EPT2 = E // NS
K = 80
EPT = E // NW
CH = EPT // K
ACC = N

f32 = jnp.float32

_mesh = plsc.VectorSubcoreMesh(
    core_axis_name="c", subcore_axis_name="s", num_cores=NC, num_subcores=NS)
_sc_params = pltpu.CompilerParams(needs_layout_passes=False)


@functools.partial(
    pl.kernel,
    out_type=jax.ShapeDtypeStruct((2, NP), f32),
    mesh=_mesh,
    scratch_types=[
        pltpu.VMEM((EPT2,), jnp.int32),
        pltpu.VMEM((NP,), f32),
        pltpu.VMEM((RPT_DEG,), f32),
        pltpu.VMEM((RPT_DEG,), f32),
        pltpu.VMEM_SHARED((NS, NP), f32),
    ],
    compiler_params=_sc_params,
)
def _deg_kernel(idx_hbm, deg_out, idx_v, deg_v, acc_v, tmp_v, deg_sh):
    c = lax.axis_index("c")
    s = lax.axis_index("s")
    row = c * NS + s
    pltpu.sync_copy(idx_hbm.at[row], idx_v)

    zero16 = jnp.zeros((16,), f32)
    ones16 = jnp.ones((16,), f32)

    def zbody(i, carry):
        deg_v[pl.ds(i * 16, 16)] = zero16
        return carry
    lax.fori_loop(0, NP // 16, zbody, None)

    def ebody(e, carry):
        idx = idx_v[pl.ds(e * 16, 16)]
        plsc.addupdate_scatter(deg_v, [idx], ones16)
        return carry
    lax.fori_loop(0, EPT2 // 16, ebody, None)

    pltpu.sync_copy(deg_v, deg_sh.at[s])
    plsc.subcore_barrier()

    base = s * RPT_DEG
    pltpu.sync_copy(deg_sh.at[0, pl.ds(base, RPT_DEG)], acc_v)
    for p in range(1, NS):
        pltpu.sync_copy(deg_sh.at[p, pl.ds(base, RPT_DEG)], tmp_v)

        def abody(i, carry):
            sl = pl.ds(i * 16, 16)
            acc_v[sl] = acc_v[sl] + tmp_v[sl]
            return carry
        lax.fori_loop(0, RPT_DEG // 16, abody, None)
    pltpu.sync_copy(acc_v, deg_out.at[c, pl.ds(base, RPT_DEG)])


@functools.partial(
    pl.kernel,
    out_type=jax.ShapeDtypeStruct((NC, N, D), f32),
    mesh=_mesh,
    scratch_types=[
        pltpu.VMEM((EPT,), jnp.int32),
        pltpu.VMEM((EPT,), jnp.int32),
        pltpu.VMEM((K, D), f32),
        pltpu.VMEM((K, D), f32),
        pltpu.VMEM_SHARED((ACC, D), f32),
        pltpu.SemaphoreType.DMA,
        pltpu.SemaphoreType.DMA,
    ],
    compiler_params=_sc_params,
)
def _scatter_kernel(src_hbm, dst_hbm, h_hbm, out_hbm,
                    src_v, dst_v, rows_a, rows_b, acc_sh, sem_a, sem_b):
    c = lax.axis_index("c")
    s = lax.axis_index("s")
    w = c * NS + s
    pltpu.sync_copy(src_hbm.at[w], src_v)
    pltpu.sync_copy(dst_hbm.at[w], dst_v)

    zero16 = jnp.zeros((16,), f32)

    def zrow(i, carry):
        for j in range(D // 16):
            rows_a[i, pl.ds(j * 16, 16)] = zero16
        return carry
    lax.fori_loop(0, K, zrow, None)

    for i in range(-(-(ACC // K) // NS)):
        m = i * NS + s

        @pl.when(m < ACC // K)
        def _():
            pltpu.sync_copy(rows_a, acc_sh.at[pl.ds(m * K, K)])
    plsc.subcore_barrier()

    pltpu.async_copy(h_hbm.at[src_v.at[pl.ds(0, K)]], rows_a, sem_a)

    def pair(j2, carry):
        j = j2 * 2 * K
        pltpu.make_async_copy(
            h_hbm.at[src_v.at[pl.ds(j, K)]], rows_a, sem_a).wait()
        pltpu.async_copy(h_hbm.at[src_v.at[pl.ds(j + K, K)]], rows_b, sem_b)
        pltpu.sync_copy(rows_a, acc_sh.at[dst_v.at[pl.ds(j, K)]], add=True)
        pltpu.make_async_copy(
            h_hbm.at[src_v.at[pl.ds(j + K, K)]], rows_b, sem_b).wait()
        pltpu.async_copy(h_hbm.at[src_v.at[pl.ds(j + 2 * K, K)]], rows_a,
                         sem_a)
        pltpu.sync_copy(rows_b, acc_sh.at[dst_v.at[pl.ds(j + K, K)]],
                        add=True)
        return carry
    lax.fori_loop(0, CH // 2, pair, None)
    pltpu.make_async_copy(
        h_hbm.at[src_v.at[pl.ds((CH - 1) * K, K)]], rows_a, sem_a).wait()
    pltpu.sync_copy(rows_a, acc_sh.at[dst_v.at[pl.ds((CH - 1) * K, K)]],
                    add=True)

    plsc.subcore_barrier()

    for i in range(-(-(ACC // K) // NS)):
        m = i * NS + s

        @pl.when(m < ACC // K)
        def _(m=m):
            pltpu.sync_copy(acc_sh.at[pl.ds(m * K, K)], rows_a)
            pltpu.sync_copy(rows_a, out_hbm.at[c, pl.ds(m * K, K)])


_BLK = 2000
_GRID = N // _BLK


def _mm_scale_body(x_ref, w_ref, b_ref, degs_ref, o_ref):
    h = jnp.dot(x_ref[...], w_ref[...], preferred_element_type=f32) + b_ref[...]
    o_ref[...] = h * lax.rsqrt(jnp.maximum(degs_ref[...], 1.0))


def _tc_mm_scale(x, w, b2d, degs):
    return pl.pallas_call(
        _mm_scale_body,
        grid=(_GRID,),
        in_specs=[
            pl.BlockSpec((_BLK, D), lambda i: (i, 0)),
            pl.BlockSpec((D, D), lambda i: (0, 0)),
            pl.BlockSpec((1, D), lambda i: (0, 0)),
            pl.BlockSpec((_BLK, 1), lambda i: (i, 0)),
        ],
        out_specs=pl.BlockSpec((_BLK, D), lambda i: (i, 0)),
        out_shape=jax.ShapeDtypeStruct((N, D), f32),
    )(x, w, b2d, degs)


def _post_body(p_ref, degd_ref, xsum_ref, x_ref, o_ref):
    a = (p_ref[0] + p_ref[1]) * lax.rsqrt(jnp.maximum(degd_ref[...], 1.0))
    xn = jnp.where(a >= 0, a, 0.01 * a)
    x_ref[...] = xn
    o_ref[...] = xsum_ref[...] + xn


def _tc_post(p, degd, xsum):
    return pl.pallas_call(
        _post_body,
        grid=(_GRID,),
        in_specs=[
            pl.BlockSpec((NC, _BLK, D), lambda i: (0, i, 0)),
            pl.BlockSpec((_BLK, 1), lambda i: (i, 0)),
            pl.BlockSpec((_BLK, D), lambda i: (i, 0)),
        ],
        out_specs=[
            pl.BlockSpec((_BLK, D), lambda i: (i, 0)),
            pl.BlockSpec((_BLK, D), lambda i: (i, 0)),
        ],
        out_shape=[
            jax.ShapeDtypeStruct((N, D), f32),
            jax.ShapeDtypeStruct((N, D), f32),
        ],
    )(p, degd, xsum)


def _fin_body(x0_ref, xsum_ref, o_ref):
    o_ref[...] = (x0_ref[...] + xsum_ref[...]) * (1.0 / 3.0)


def _tc_fin(x0, xsum):
    return pl.pallas_call(
        _fin_body,
        grid=(_GRID,),
        in_specs=[
            pl.BlockSpec((_BLK, D), lambda i: (i, 0)),
            pl.BlockSpec((_BLK, D), lambda i: (i, 0)),
        ],
        out_specs=pl.BlockSpec((_BLK, D), lambda i: (i, 0)),
        out_shape=jax.ShapeDtypeStruct((N, D), f32),
    )(x0, xsum)


def kernel(edge_index, all_embed, W1, b1, W2, b2):
    ei = edge_index.astype(jnp.int32)
    deg_idx = ei.reshape(NW, EPT2)
    src_r = ei[0].reshape(NW, EPT)
    dst_r = ei[1].reshape(NW, EPT)

    degs = _deg_kernel(deg_idx)
    deg_src = degs[0, :N].reshape(N, 1)
    deg_dst = degs[1, :N].reshape(N, 1)
    Ws = jnp.stack((W1, W2))
    bs = jnp.stack((b1.reshape(1, D), b2.reshape(1, D)))

    nlayers = jnp.int32(2) + jnp.min(ei[0]) * jnp.int32(0)

    def cond(carry):
        i, _, _, _ = carry
        return i < nlayers

    def body(carry):
        i, x, x1, xsum = carry
        w = lax.dynamic_index_in_dim(Ws, i, axis=0, keepdims=False)
        b2d = lax.dynamic_index_in_dim(bs, i, axis=0, keepdims=False)
        h = _tc_mm_scale(x, w, b2d, deg_src)
        p = _scatter_kernel(src_r, dst_r, h)
        xn, xsumn = _tc_post(p, deg_dst, xsum)
        return (i + 1, xn, x, xsumn)

    z = jnp.zeros((N, D), f32)
    _, _, _, xsum = lax.while_loop(
        cond, body, (jnp.int32(0), all_embed, z, z))
    return _tc_fin(all_embed, xsum)


# speedup vs baseline: 3.0553x; 1.0228x over previous
import functools

import jax
import jax.numpy as jnp
from jax import lax
from jax.experimental import pallas as pl
from jax.experimental.pallas import tpu as pltpu
from jax.experimental.pallas import tpu_sc as plsc

N = 10000
E = 320000
D = 128
NC = 2
NS = 16
NW = NC * NS
NP = 10240
RPT_DEG = NP // NS
EPT2 = E // NS
K = 80
EPT = E // NW
CH = EPT // K
ACC = N

f32 = jnp.float32

_mesh = plsc.VectorSubcoreMesh(
    core_axis_name="c", subcore_axis_name="s", num_cores=NC, num_subcores=NS)
_sc_params = pltpu.CompilerParams(needs_layout_passes=False)


@functools.partial(
    pl.kernel,
    out_type=jax.ShapeDtypeStruct((2, NP), f32),
    mesh=_mesh,
    scratch_types=[
        pltpu.VMEM((EPT2,), jnp.int32),
        pltpu.VMEM((NP,), f32),
        pltpu.VMEM((RPT_DEG,), f32),
        pltpu.VMEM((RPT_DEG,), f32),
        pltpu.VMEM_SHARED((NS, NP), f32),
    ],
    compiler_params=_sc_params,
)
def _deg_kernel(idx_hbm, deg_out, idx_v, deg_v, acc_v, tmp_v, deg_sh):
    c = lax.axis_index("c")
    s = lax.axis_index("s")
    row = c * NS + s
    pltpu.sync_copy(idx_hbm.at[row], idx_v)

    zero16 = jnp.zeros((16,), f32)
    ones16 = jnp.ones((16,), f32)

    def zbody(i, carry):
        deg_v[pl.ds(i * 16, 16)] = zero16
        return carry
    lax.fori_loop(0, NP // 16, zbody, None)

    def ebody(e, carry):
        idx = idx_v[pl.ds(e * 16, 16)]
        plsc.addupdate_scatter(deg_v, [idx], ones16)
        return carry
    lax.fori_loop(0, EPT2 // 16, ebody, None)

    pltpu.sync_copy(deg_v, deg_sh.at[s])
    plsc.subcore_barrier()

    base = s * RPT_DEG
    pltpu.sync_copy(deg_sh.at[0, pl.ds(base, RPT_DEG)], acc_v)
    for p in range(1, NS):
        pltpu.sync_copy(deg_sh.at[p, pl.ds(base, RPT_DEG)], tmp_v)

        def abody(i, carry):
            sl = pl.ds(i * 16, 16)
            acc_v[sl] = acc_v[sl] + tmp_v[sl]
            return carry
        lax.fori_loop(0, RPT_DEG // 16, abody, None)
    pltpu.sync_copy(acc_v, deg_out.at[c, pl.ds(base, RPT_DEG)])


@functools.partial(
    pl.kernel,
    out_type=jax.ShapeDtypeStruct((NC, N, D), f32),
    mesh=_mesh,
    scratch_types=[
        pltpu.VMEM((EPT,), jnp.int32),
        pltpu.VMEM((EPT,), jnp.int32),
        pltpu.VMEM((K, D), f32),
        pltpu.VMEM((K, D), f32),
        pltpu.VMEM_SHARED((ACC, D), f32),
        pltpu.SemaphoreType.DMA,
        pltpu.SemaphoreType.DMA,
    ],
    compiler_params=_sc_params,
)
def _scatter_kernel(src_hbm, dst_hbm, h_hbm, out_hbm,
                    src_v, dst_v, rows_a, rows_b, acc_sh, sem_a, sem_b):
    c = lax.axis_index("c")
    s = lax.axis_index("s")
    w = c * NS + s
    pltpu.sync_copy(src_hbm.at[w], src_v)
    pltpu.sync_copy(dst_hbm.at[w], dst_v)

    zero16 = jnp.zeros((16,), f32)

    def zrow(i, carry):
        for j in range(D // 16):
            rows_a[i, pl.ds(j * 16, 16)] = zero16
        return carry
    lax.fori_loop(0, K, zrow, None)

    for i in range(-(-(ACC // K) // NS)):
        m = i * NS + s

        @pl.when(m < ACC // K)
        def _():
            pltpu.sync_copy(rows_a, acc_sh.at[pl.ds(m * K, K)])
    plsc.subcore_barrier()

    pltpu.async_copy(h_hbm.at[src_v.at[pl.ds(0, K)]], rows_a, sem_a)

    def pair(j2, carry):
        j = j2 * 2 * K
        pltpu.make_async_copy(
            h_hbm.at[src_v.at[pl.ds(j, K)]], rows_a, sem_a).wait()
        pltpu.async_copy(h_hbm.at[src_v.at[pl.ds(j + K, K)]], rows_b, sem_b)
        pltpu.sync_copy(rows_a, acc_sh.at[dst_v.at[pl.ds(j, K)]], add=True)
        pltpu.make_async_copy(
            h_hbm.at[src_v.at[pl.ds(j + K, K)]], rows_b, sem_b).wait()
        pltpu.async_copy(h_hbm.at[src_v.at[pl.ds(j + 2 * K, K)]], rows_a,
                         sem_a)
        pltpu.sync_copy(rows_b, acc_sh.at[dst_v.at[pl.ds(j + K, K)]],
                        add=True)
        return carry
    lax.fori_loop(0, CH // 2, pair, None)
    pltpu.make_async_copy(
        h_hbm.at[src_v.at[pl.ds((CH - 1) * K, K)]], rows_a, sem_a).wait()
    pltpu.sync_copy(rows_a, acc_sh.at[dst_v.at[pl.ds((CH - 1) * K, K)]],
                    add=True)

    plsc.subcore_barrier()

    for i in range(-(-(ACC // K) // NS)):
        m = i * NS + s

        @pl.when(m < ACC // K)
        def _(m=m):
            pltpu.sync_copy(acc_sh.at[pl.ds(m * K, K)], rows_a)
            pltpu.sync_copy(rows_a, out_hbm.at[c, pl.ds(m * K, K)])


_BLK = 2000
_GRID = N // _BLK


def _mm_scale_body(x_ref, w_ref, b_ref, degs_ref, o_ref):
    h = jnp.dot(x_ref[...], w_ref[...], preferred_element_type=f32) + b_ref[...]
    o_ref[...] = h * lax.rsqrt(jnp.maximum(degs_ref[...], 1.0))


def _tc_mm_scale(x, w, b2d, degs):
    return pl.pallas_call(
        _mm_scale_body,
        grid=(_GRID,),
        in_specs=[
            pl.BlockSpec((_BLK, D), lambda i: (i, 0)),
            pl.BlockSpec((D, D), lambda i: (0, 0)),
            pl.BlockSpec((1, D), lambda i: (0, 0)),
            pl.BlockSpec((_BLK, 1), lambda i: (i, 0)),
        ],
        out_specs=pl.BlockSpec((_BLK, D), lambda i: (i, 0)),
        out_shape=jax.ShapeDtypeStruct((N, D), f32),
    )(x, w, b2d, degs)


def _step_body(p_ref, degd_ref, w_ref, b_ref, degs_ref, xsum_ref,
               xsum_out_ref, h_ref):
    a = (p_ref[0] + p_ref[1]) * lax.rsqrt(jnp.maximum(degd_ref[...], 1.0))
    xn = jnp.where(a >= 0, a, 0.01 * a)
    xsum_out_ref[...] = xsum_ref[...] + xn
    h = jnp.dot(xn, w_ref[...], preferred_element_type=f32) + b_ref[...]
    h_ref[...] = h * lax.rsqrt(jnp.maximum(degs_ref[...], 1.0))


def _tc_step(p, degd, w, b2d, degs, xsum):
    return pl.pallas_call(
        _step_body,
        grid=(_GRID,),
        in_specs=[
            pl.BlockSpec((NC, _BLK, D), lambda i: (0, i, 0)),
            pl.BlockSpec((_BLK, 1), lambda i: (i, 0)),
            pl.BlockSpec((D, D), lambda i: (0, 0)),
            pl.BlockSpec((1, D), lambda i: (0, 0)),
            pl.BlockSpec((_BLK, 1), lambda i: (i, 0)),
            pl.BlockSpec((_BLK, D), lambda i: (i, 0)),
        ],
        out_specs=[
            pl.BlockSpec((_BLK, D), lambda i: (i, 0)),
            pl.BlockSpec((_BLK, D), lambda i: (i, 0)),
        ],
        out_shape=[
            jax.ShapeDtypeStruct((N, D), f32),
            jax.ShapeDtypeStruct((N, D), f32),
        ],
    )(p, degd, w, b2d, degs, xsum)


def _fin_body(x0_ref, xsum_ref, o_ref):
    o_ref[...] = (x0_ref[...] + xsum_ref[...]) * (1.0 / 3.0)


def _tc_fin(x0, xsum):
    return pl.pallas_call(
        _fin_body,
        grid=(_GRID,),
        in_specs=[
            pl.BlockSpec((_BLK, D), lambda i: (i, 0)),
            pl.BlockSpec((_BLK, D), lambda i: (i, 0)),
        ],
        out_specs=pl.BlockSpec((_BLK, D), lambda i: (i, 0)),
        out_shape=jax.ShapeDtypeStruct((N, D), f32),
    )(x0, xsum)


def kernel(edge_index, all_embed, W1, b1, W2, b2):
    ei = edge_index.astype(jnp.int32)
    deg_idx = ei.reshape(NW, EPT2)
    src_r = ei[0].reshape(NW, EPT)
    dst_r = ei[1].reshape(NW, EPT)

    degs = _deg_kernel(deg_idx)
    deg_src = degs[0, :N].reshape(N, 1)
    deg_dst = degs[1, :N].reshape(N, 1)
    Ws = jnp.stack((W1, W2))
    bs = jnp.stack((b1.reshape(1, D), b2.reshape(1, D)))

    nlayers = jnp.int32(2) + jnp.min(ei[0]) * jnp.int32(0)

    def cond(carry):
        i, _, _ = carry
        return i < nlayers

    def body(carry):
        i, h, xsum = carry
        p = _scatter_kernel(src_r, dst_r, h)
        inext = jnp.minimum(i + 1, 1)
        w = lax.dynamic_index_in_dim(Ws, inext, axis=0, keepdims=False)
        b2d = lax.dynamic_index_in_dim(bs, inext, axis=0, keepdims=False)
        xsumn, hn = _tc_step(p, deg_dst, w, b2d, deg_src, xsum)
        return (i + 1, hn, xsumn)

    h0 = _tc_mm_scale(all_embed, W1, bs[0], deg_src)
    z = jnp.zeros((N, D), f32)
    _, _, xsum = lax.while_loop(cond, body, (jnp.int32(0), h0, z))
    return _tc_fin(all_embed, xsum)
